# Initial kernel scaffold; baseline (speedup 1.0000x reference)
import jax
import jax.numpy as jnp
from jax.experimental import pallas as pl


def kernel(h, edge_index, he, W1, b1, W2, b2, W3, b3, W4, b4):
    raise NotImplementedError("write your pallas kernel here")


# speedup vs baseline: 2.9253x; 2.9253x over previous
import functools

import jax
import jax.numpy as jnp
from jax import lax
from jax.experimental import pallas as pl
from jax.experimental.pallas import tpu as pltpu
from jax.experimental.pallas import tpu_sc as plsc

_NC = 2
_NS = 16
_NW = _NC * _NS
_B = 128


def _mlp2_body(x_ref, wa_ref, ba_ref, wb_ref, bb_ref, o_ref):
    x = x_ref[:]
    y = lax.dot_general(x, wa_ref[:], (((1,), (1,)), ((), ())),
                        preferred_element_type=jnp.float32) + ba_ref[:]
    y = jnp.maximum(y, 0.0)
    o_ref[:] = lax.dot_general(y, wb_ref[:], (((1,), (1,)), ((), ())),
                               preferred_element_type=jnp.float32) + bb_ref[:]


def _edge_mlp(he, W1, b1, W2, b2):
    E, D = he.shape
    BE = 2000
    grid = (E // BE,)
    return pl.pallas_call(
        _mlp2_body,
        grid=grid,
        in_specs=[
            pl.BlockSpec((BE, D), lambda i: (i, 0)),
            pl.BlockSpec((D, D), lambda i: (0, 0)),
            pl.BlockSpec((1, D), lambda i: (0, 0)),
            pl.BlockSpec((D, D), lambda i: (0, 0)),
            pl.BlockSpec((1, D), lambda i: (0, 0)),
        ],
        out_specs=pl.BlockSpec((BE, D), lambda i: (i, 0)),
        out_shape=jax.ShapeDtypeStruct((E, D), jnp.float32),
    )(he, W1, b1.reshape(1, D), W2, b2.reshape(1, D))


def _sum2_mlp2_body(p0_ref, p1_ref, wa_ref, ba_ref, wb_ref, bb_ref, o_ref):
    x = p0_ref[:] + p1_ref[:]
    y = lax.dot_general(x, wa_ref[:], (((1,), (1,)), ((), ())),
                        preferred_element_type=jnp.float32) + ba_ref[:]
    y = jnp.maximum(y, 0.0)
    o_ref[:] = lax.dot_general(y, wb_ref[:], (((1,), (1,)), ((), ())),
                               preferred_element_type=jnp.float32) + bb_ref[:]


def _node_mlp(partials, N, W3, b3, W4, b4):
    D = partials.shape[1]
    BN = 2000
    grid = (N // BN,)
    nb = N // BN
    return pl.pallas_call(
        _sum2_mlp2_body,
        grid=grid,
        in_specs=[
            pl.BlockSpec((BN, D), lambda i: (i, 0)),
            pl.BlockSpec((BN, D), lambda i, nb=nb: (i + nb, 0)),
            pl.BlockSpec((D, D), lambda i: (0, 0)),
            pl.BlockSpec((1, D), lambda i: (0, 0)),
            pl.BlockSpec((D, D), lambda i: (0, 0)),
            pl.BlockSpec((1, D), lambda i: (0, 0)),
        ],
        out_specs=pl.BlockSpec((BN, D), lambda i: (i, 0)),
        out_shape=jax.ShapeDtypeStruct((N, D), jnp.float32),
    )(partials, partials, W3, b3.reshape(1, D), W4, b4.reshape(1, D))


def _gather_mul_scatter(h, src, dst, he2):
    N, D = h.shape
    E = src.shape[0]
    C = E // _B
    zr = 40
    nslab = N // zr
    mesh = plsc.VectorSubcoreMesh(core_axis_name="c", subcore_axis_name="s")

    @functools.partial(
        pl.kernel,
        mesh=mesh,
        out_type=jax.ShapeDtypeStruct((_NC * N, D), jnp.float32),
        scratch_types=[
            pltpu.VMEM((_B,), jnp.int32),
            pltpu.VMEM((_B,), jnp.int32),
            pltpu.VMEM((_B, D), jnp.float32),
            pltpu.VMEM((_B, D), jnp.float32),
            pltpu.VMEM((zr, D), jnp.float32),
            pltpu.VMEM_SHARED((N, D), jnp.float32),
            pltpu.SemaphoreType.DMA,
        ],
    )
    def k(h_hbm, src_hbm, dst_hbm, he2_hbm, out_hbm,
          sidx, didx, hrows, erows, zbuf, agg, sem):
        cid = lax.axis_index("c")
        sid = lax.axis_index("s")
        wid = sid * _NC + cid

        zero = jnp.zeros((16,), jnp.float32)

        def zrow(r, carry):
            for j in range(D // 16):
                zbuf[r, pl.ds(j * 16, 16)] = zero
            return carry

        lax.fori_loop(0, zr, zrow, 0)
        ns = (nslab // _NS) + (sid < (nslab % _NS)).astype(jnp.int32)

        def zslab(t, carry):
            pltpu.sync_copy(zbuf, agg.at[pl.ds((sid + t * _NS) * zr, zr)])
            return carry

        lax.fori_loop(0, ns, zslab, 0)
        plsc.subcore_barrier()

        nk = (C // _NW) + (wid < (C % _NW)).astype(jnp.int32)

        def chunk(kk, carry):
            base = (wid + kk * _NW) * _B
            pltpu.sync_copy(src_hbm.at[pl.ds(base, _B)], sidx)
            pltpu.sync_copy(dst_hbm.at[pl.ds(base, _B)], didx)
            pltpu.async_copy(h_hbm.at[sidx], hrows, sem).wait()
            pltpu.sync_copy(he2_hbm.at[pl.ds(base, _B)], erows)

            def mrow(r, c2):
                for j in range(D // 16):
                    s = pl.ds(j * 16, 16)
                    erows[r, s] = erows[r, s] * hrows[r, s]
                return c2

            lax.fori_loop(0, _B, mrow, 0)
            pltpu.sync_copy(erows, agg.at[didx], add=True)
            return carry

        lax.fori_loop(0, nk, chunk, 0)
        plsc.subcore_barrier()

        def wslab(t, carry):
            r0 = (sid + t * _NS) * zr
            pltpu.sync_copy(agg.at[pl.ds(r0, zr)], zbuf)
            pltpu.sync_copy(zbuf, out_hbm.at[pl.ds(cid * N + r0, zr)])
            return carry

        lax.fori_loop(0, ns, wslab, 0)

    return k(h, src, dst, he2)


def kernel(h, edge_index, he, W1, b1, W2, b2, W3, b3, W4, b4):
    N, D = h.shape
    src = edge_index[0]
    dst = edge_index[1]
    he2 = _edge_mlp(he, W1, b1, W2, b2)
    partials = _gather_mul_scatter(h, src, dst, he2)
    return _node_mlp(partials, N, W3, b3, W4, b4)


# speedup vs baseline: 3.8833x; 1.3275x over previous
import functools

import jax
import jax.numpy as jnp
from jax import lax
from jax.experimental import pallas as pl
from jax.experimental.pallas import tpu as pltpu
from jax.experimental.pallas import tpu_sc as plsc

_NC = 2
_NS = 16
_NW = _NC * _NS
_B = 80


def _mlp2_body(x_ref, wa_ref, ba_ref, wb_ref, bb_ref, o_ref):
    x = x_ref[:]
    y = lax.dot_general(x, wa_ref[:], (((1,), (1,)), ((), ())),
                        preferred_element_type=jnp.float32) + ba_ref[:]
    y = jnp.maximum(y, 0.0)
    o_ref[:] = lax.dot_general(y, wb_ref[:], (((1,), (1,)), ((), ())),
                               preferred_element_type=jnp.float32) + bb_ref[:]


def _edge_mlp(he, W1, b1, W2, b2):
    E, D = he.shape
    BE = 2000
    grid = (E // BE,)
    return pl.pallas_call(
        _mlp2_body,
        grid=grid,
        in_specs=[
            pl.BlockSpec((BE, D), lambda i: (i, 0)),
            pl.BlockSpec((D, D), lambda i: (0, 0)),
            pl.BlockSpec((1, D), lambda i: (0, 0)),
            pl.BlockSpec((D, D), lambda i: (0, 0)),
            pl.BlockSpec((1, D), lambda i: (0, 0)),
        ],
        out_specs=pl.BlockSpec((BE, D), lambda i: (i, 0)),
        out_shape=jax.ShapeDtypeStruct((E, D), jnp.float32),
    )(he, W1, b1.reshape(1, D), W2, b2.reshape(1, D))


def _sum2_mlp2_body(p0_ref, p1_ref, wa_ref, ba_ref, wb_ref, bb_ref, o_ref):
    x = p0_ref[:] + p1_ref[:]
    y = lax.dot_general(x, wa_ref[:], (((1,), (1,)), ((), ())),
                        preferred_element_type=jnp.float32) + ba_ref[:]
    y = jnp.maximum(y, 0.0)
    o_ref[:] = lax.dot_general(y, wb_ref[:], (((1,), (1,)), ((), ())),
                               preferred_element_type=jnp.float32) + bb_ref[:]


def _node_mlp(partials, N, W3, b3, W4, b4):
    D = partials.shape[1]
    BN = 2000
    grid = (N // BN,)
    nb = N // BN
    return pl.pallas_call(
        _sum2_mlp2_body,
        grid=grid,
        in_specs=[
            pl.BlockSpec((BN, D), lambda i: (i, 0)),
            pl.BlockSpec((BN, D), lambda i, nb=nb: (i + nb, 0)),
            pl.BlockSpec((D, D), lambda i: (0, 0)),
            pl.BlockSpec((1, D), lambda i: (0, 0)),
            pl.BlockSpec((D, D), lambda i: (0, 0)),
            pl.BlockSpec((1, D), lambda i: (0, 0)),
        ],
        out_specs=pl.BlockSpec((BN, D), lambda i: (i, 0)),
        out_shape=jax.ShapeDtypeStruct((N, D), jnp.float32),
    )(partials, partials, W3, b3.reshape(1, D), W4, b4.reshape(1, D))


def _gather_mul_scatter(h, src, dst, he2):
    N, D = h.shape
    E = src.shape[0]
    M = E // (_B * _NW)
    zr = 40
    nslab = N // zr
    mesh = plsc.VectorSubcoreMesh(core_axis_name="c", subcore_axis_name="s")

    @functools.partial(
        pl.kernel,
        mesh=mesh,
        out_type=jax.ShapeDtypeStruct((_NC * N, D), jnp.float32),
        scratch_types=[
            pltpu.VMEM((_B,), jnp.int32),
            pltpu.VMEM((_B,), jnp.int32),
            pltpu.VMEM((_B,), jnp.int32),
            pltpu.VMEM((_B,), jnp.int32),
            pltpu.VMEM((_B, D), jnp.float32),
            pltpu.VMEM((_B, D), jnp.float32),
            pltpu.VMEM((_B, D), jnp.float32),
            pltpu.VMEM((_B, D), jnp.float32),
            pltpu.VMEM((zr, D), jnp.float32),
            pltpu.VMEM_SHARED((N, D), jnp.float32),
            pltpu.SemaphoreType.DMA,
            pltpu.SemaphoreType.DMA,
            pltpu.SemaphoreType.DMA,
            pltpu.SemaphoreType.DMA,
        ],
    )
    def k(h_hbm, src_hbm, dst_hbm, he2_hbm, out_hbm,
          sidx0, sidx1, didx0, didx1, hrows0, hrows1, erows0, erows1,
          zbuf, agg, g0, g1, e0, e1):
        cid = lax.axis_index("c")
        sid = lax.axis_index("s")
        wid = sid * _NC + cid

        sidx = (sidx0, sidx1)
        didx = (didx0, didx1)
        hrows = (hrows0, hrows1)
        erows = (erows0, erows1)
        gsem = (g0, g1)
        esem = (e0, e1)

        zero = jnp.zeros((16,), jnp.float32)

        def zrow(r, carry):
            for j in range(D // 16):
                zbuf[r, pl.ds(j * 16, 16)] = zero
            return carry

        lax.fori_loop(0, zr, zrow, 0)
        ns = (nslab // _NS) + (sid < (nslab % _NS)).astype(jnp.int32)

        def zslab(t, carry):
            pltpu.sync_copy(zbuf, agg.at[pl.ds((sid + t * _NS) * zr, zr)])
            return carry

        lax.fori_loop(0, ns, zslab, 0)
        plsc.subcore_barrier()

        def load(kk, b):
            base = (wid * M + kk) * _B
            pltpu.sync_copy(src_hbm.at[pl.ds(base, _B)], sidx[b])
            pltpu.sync_copy(dst_hbm.at[pl.ds(base, _B)], didx[b])
            pltpu.async_copy(h_hbm.at[sidx[b]], hrows[b], gsem[b])
            pltpu.async_copy(he2_hbm.at[pl.ds(base, _B)], erows[b], esem[b])

        def compute(kk, b):
            base = (wid * M + kk) * _B
            pltpu.make_async_copy(h_hbm.at[sidx[b]], hrows[b], gsem[b]).wait()
            pltpu.make_async_copy(
                he2_hbm.at[pl.ds(base, _B)], erows[b], esem[b]).wait()

            def mrow(r, c2):
                for j in range(D // 16):
                    s = pl.ds(j * 16, 16)
                    erows[b][r, s] = erows[b][r, s] * hrows[b][r, s]
                return c2

            lax.fori_loop(0, _B, mrow, 0)
            pltpu.sync_copy(erows[b], agg.at[didx[b]], add=True)

        load(0, 0)

        def pair(j, carry):
            kk = 2 * j
            load(kk + 1, 1)
            compute(kk, 0)
            load(kk + 2, 0)
            compute(kk + 1, 1)
            return carry

        lax.fori_loop(0, (M - 1) // 2, pair, 0)
        compute(M - 1, (M - 1) % 2)
        plsc.subcore_barrier()

        def wslab(t, carry):
            r0 = (sid + t * _NS) * zr
            pltpu.sync_copy(agg.at[pl.ds(r0, zr)], zbuf)
            pltpu.sync_copy(zbuf, out_hbm.at[pl.ds(cid * N + r0, zr)])
            return carry

        lax.fori_loop(0, ns, wslab, 0)

    return k(h, src, dst, he2)


def kernel(h, edge_index, he, W1, b1, W2, b2, W3, b3, W4, b4):
    N, D = h.shape
    src = edge_index[0]
    dst = edge_index[1]
    he2 = _edge_mlp(he, W1, b1, W2, b2)
    partials = _gather_mul_scatter(h, src, dst, he2)
    return _node_mlp(partials, N, W3, b3, W4, b4)


# speedup vs baseline: 4.0886x; 1.0529x over previous
import functools

import jax
import jax.numpy as jnp
from jax import lax
from jax.experimental import pallas as pl
from jax.experimental.pallas import tpu as pltpu
from jax.experimental.pallas import tpu_sc as plsc

_NC = 2
_NS = 16
_NW = _NC * _NS
_B = 80


def _mlp2_body(x_ref, wa_ref, ba_ref, wb_ref, bb_ref, o_ref):
    x = x_ref[:]
    y = lax.dot_general(x, wa_ref[:], (((1,), (1,)), ((), ())),
                        preferred_element_type=jnp.float32) + ba_ref[:]
    y = jnp.maximum(y, 0.0)
    y = lax.dot_general(y, wb_ref[:], (((1,), (1,)), ((), ())),
                        preferred_element_type=jnp.float32) + bb_ref[:]
    yb = y.astype(jnp.bfloat16)
    h2 = yb.shape[1] // 2
    lo = lax.convert_element_type(
        lax.bitcast_convert_type(yb[:, :h2], jnp.uint16), jnp.uint32)
    hi = lax.convert_element_type(
        lax.bitcast_convert_type(yb[:, h2:], jnp.uint16), jnp.uint32)
    o_ref[:] = lax.bitcast_convert_type((hi << 16) | lo, jnp.int32)


def _edge_mlp(he, W1, b1, W2, b2):
    E, D = he.shape
    BE = 2560
    grid = (E // BE,)
    return pl.pallas_call(
        _mlp2_body,
        grid=grid,
        in_specs=[
            pl.BlockSpec((BE, D), lambda i: (i, 0)),
            pl.BlockSpec((D, D), lambda i: (0, 0)),
            pl.BlockSpec((1, D), lambda i: (0, 0)),
            pl.BlockSpec((D, D), lambda i: (0, 0)),
            pl.BlockSpec((1, D), lambda i: (0, 0)),
        ],
        out_specs=pl.BlockSpec((BE, D // 2), lambda i: (i, 0)),
        out_shape=jax.ShapeDtypeStruct((E, D // 2), jnp.int32),
    )(he, W1, b1.reshape(1, D), W2, b2.reshape(1, D))


def _sum4_mlp2_body(p0_ref, p1_ref, p2_ref, p3_ref,
                    wa_ref, ba_ref, wb_ref, bb_ref, o_ref):
    x = (p0_ref[0] + p1_ref[0]) + (p2_ref[0] + p3_ref[0])
    y = lax.dot_general(x, wa_ref[:], (((1,), (1,)), ((), ())),
                        preferred_element_type=jnp.float32) + ba_ref[:]
    y = jnp.maximum(y, 0.0)
    o_ref[:] = lax.dot_general(y, wb_ref[:], (((1,), (1,)), ((), ())),
                               preferred_element_type=jnp.float32) + bb_ref[:]


def _node_mlp(pa, pb, N, W3, b3, W4, b4):
    D = pa.shape[2]
    BN = 2000
    grid = (N // BN,)
    pspec = [
        pl.BlockSpec((1, BN, D), lambda i: (0, i, 0)),
        pl.BlockSpec((1, BN, D), lambda i: (1, i, 0)),
    ]
    return pl.pallas_call(
        _sum4_mlp2_body,
        grid=grid,
        in_specs=pspec + pspec + [
            pl.BlockSpec((D, D), lambda i: (0, 0)),
            pl.BlockSpec((1, D), lambda i: (0, 0)),
            pl.BlockSpec((D, D), lambda i: (0, 0)),
            pl.BlockSpec((1, D), lambda i: (0, 0)),
        ],
        out_specs=pl.BlockSpec((BN, D), lambda i: (i, 0)),
        out_shape=jax.ShapeDtypeStruct((N, D), jnp.float32),
    )(pa, pa, pb, pb, W3, b3.reshape(1, D), W4, b4.reshape(1, D))


def _gather_mul_scatter(h, eidx, he2):
    N, D = h.shape
    Np = 10240
    C = eidx.shape[0]
    M = C // _NW
    zr = 40
    wr = 80
    mesh = plsc.VectorSubcoreMesh(core_axis_name="c", subcore_axis_name="s")

    @functools.partial(
        pl.kernel,
        mesh=mesh,
        out_type=jax.ShapeDtypeStruct((_NC * Np, D), jnp.float32),
        scratch_types=[
            pltpu.VMEM((2, _B), jnp.int32),
            pltpu.VMEM((2, _B), jnp.int32),
            pltpu.VMEM((_B, D), jnp.float32),
            pltpu.VMEM((_B, D), jnp.float32),
            pltpu.VMEM((_B, D // 2), jnp.int32),
            pltpu.VMEM((_B, D // 2), jnp.int32),
            pltpu.VMEM((zr, D), jnp.float32),
            pltpu.VMEM_SHARED((Np, D), jnp.float32),
            pltpu.SemaphoreType.DMA,
            pltpu.SemaphoreType.DMA,
            pltpu.SemaphoreType.DMA,
            pltpu.SemaphoreType.DMA,
            pltpu.SemaphoreType.DMA,
            pltpu.SemaphoreType.DMA,
            pltpu.SemaphoreType.DMA,
            pltpu.SemaphoreType.DMA,
            pltpu.SemaphoreType.DMA,
        ],
    )
    def k(h_hbm, eidx_hbm, he2_hbm, out_hbm,
          ibuf0, ibuf1, hrows0, hrows1, erows0, erows1,
          zbuf, agg, g0, g1, e0, e1, s0, s1, zsem, w0, w1):
        cid = lax.axis_index("c")
        sid = lax.axis_index("s")
        wid = sid * _NC + cid

        ibuf = (ibuf0, ibuf1)
        hrows = (hrows0, hrows1)
        erows = (erows0, erows1)
        gsem = (g0, g1)
        esem = (e0, e1)
        ssem = (s0, s1)
        wsem = (w0, w1)

        zero = jnp.zeros((16,), jnp.float32)

        def zrow(r, carry):
            for j in range(D // 16):
                zbuf[r, pl.ds(j * 16, 16)] = zero
            return carry

        lax.fori_loop(0, zr, zrow, 0)

        def zslab(t, carry):
            pltpu.async_copy(zbuf, agg.at[pl.ds((sid + t * _NS) * zr, zr)],
                             zsem)
            return carry

        nz = Np // (zr * _NS)
        lax.fori_loop(0, nz, zslab, 0)

        def zdrain(t, carry):
            pltpu.make_async_copy(
                zbuf, agg.at[pl.ds((sid + t * _NS) * zr, zr)], zsem).wait()
            return carry

        lax.fori_loop(0, nz, zdrain, 0)
        plsc.subcore_barrier()

        def scatter_wait(b):
            pltpu.make_async_copy(hrows[b], agg.at[ibuf[b].at[1]],
                                  ssem[b]).wait()

        def load(kk, b):
            ck = wid * M + kk

            @pl.when(kk >= 2)
            def _():
                scatter_wait(b)

            pltpu.sync_copy(eidx_hbm.at[ck], ibuf[b])
            pltpu.async_copy(h_hbm.at[ibuf[b].at[0]], hrows[b], gsem[b])
            pltpu.async_copy(he2_hbm.at[pl.ds(ck * _B, _B)], erows[b], esem[b])

        def compute(kk, b):
            ck = wid * M + kk
            pltpu.make_async_copy(
                h_hbm.at[ibuf[b].at[0]], hrows[b], gsem[b]).wait()
            pltpu.make_async_copy(
                he2_hbm.at[pl.ds(ck * _B, _B)], erows[b], esem[b]).wait()

            @plsc.parallel_loop(0, _B, step=1, unroll=4)
            def mrow(r):
                for j in range(D // 32):
                    ew = erows[b][r, pl.ds(16 * j, 16)]
                    ea = lax.bitcast_convert_type(ew << 16, jnp.float32)
                    eo = lax.bitcast_convert_type(ew & jnp.int32(-65536),
                                                  jnp.float32)
                    s0 = pl.ds(16 * j, 16)
                    s1 = pl.ds(D // 2 + 16 * j, 16)
                    hrows[b][r, s0] = hrows[b][r, s0] * ea
                    hrows[b][r, s1] = hrows[b][r, s1] * eo

            pltpu.async_copy(hrows[b], agg.at[ibuf[b].at[1]], ssem[b],
                             add=True)

        load(0, 0)

        def pair(j, carry):
            kk = 2 * j
            load(kk + 1, 1)
            compute(kk, 0)
            load(kk + 2, 0)
            compute(kk + 1, 1)
            return carry

        if M % 2 == 1:
            lax.fori_loop(0, (M - 1) // 2, pair, 0)
            compute(M - 1, 0)
        else:
            lax.fori_loop(0, M // 2 - 1, pair, 0)
            load(M - 1, 1)
            compute(M - 2, 0)
            compute(M - 1, 1)
        scatter_wait(0)
        scatter_wait(1)
        plsc.subcore_barrier()

        def wb_issue(t, b):
            r0 = sid * (8 * wr) + t * wr
            pltpu.sync_copy(agg.at[pl.ds(r0, wr)], hrows[b])
            pltpu.async_copy(hrows[b], out_hbm.at[pl.ds(cid * Np + r0, wr)],
                             wsem[b])

        def wb_wait(t, b):
            r0 = sid * (8 * wr) + t * wr
            pltpu.make_async_copy(
                hrows[b], out_hbm.at[pl.ds(cid * Np + r0, wr)],
                wsem[b]).wait()

        for q in range(4):
            if q >= 1:
                wb_wait(2 * q - 2, 0)
            wb_issue(2 * q, 0)
            if q >= 1:
                wb_wait(2 * q - 1, 1)
            wb_issue(2 * q + 1, 1)
        wb_wait(6, 0)
        wb_wait(7, 1)

    return k(h, eidx, he2)


def kernel(h, edge_index, he, W1, b1, W2, b2, W3, b3, W4, b4):
    N, D = h.shape
    E = he.shape[0]
    eidx = jnp.stack([edge_index[0].reshape(-1, _B),
                      edge_index[1].reshape(-1, _B)], axis=1)
    C1 = 2048
    E1 = C1 * _B
    he2a = _edge_mlp(he[:E1], W1, b1, W2, b2)
    he2b = _edge_mlp(he[E1:], W1, b1, W2, b2)
    pa = _gather_mul_scatter(h, eidx[:C1], he2a)
    pb = _gather_mul_scatter(h, eidx[C1:], he2b)
    Np = pa.shape[0] // _NC
    return _node_mlp(pa.reshape(_NC, Np, D), pb.reshape(_NC, Np, D),
                     N, W3, b3, W4, b4)
